# baseline (speedup 1.0000x reference)
import functools

import jax
import jax.numpy as jnp
from jax import lax
from jax.experimental import pallas as pl
from jax.experimental.pallas import tpu as pltpu
from jax.experimental.pallas import tpu_sc as plsc

_KHOT = 128
_RCOLD = 16
_NPAIR = _RCOLD // 2
_NG = _NPAIR // 2
_D = 64
_NC, _NS, _L = 2, 16, 16
_NW = _NC * _NS
_TBL = 1 << 20


def _sc_map(tokens_flat, old_to_new):
    n = tokens_flat.shape[0]
    per_w = n // _NW
    mesh = plsc.VectorSubcoreMesh(core_axis_name="c", subcore_axis_name="s")

    @functools.partial(
        pl.kernel,
        out_type=(
            jax.ShapeDtypeStruct((n,), jnp.int32),
            jax.ShapeDtypeStruct((n,), jnp.int32),
        ),
        mesh=mesh,
        scratch_types=[
            pltpu.VMEM((per_w,), jnp.int32),
            pltpu.VMEM((per_w,), jnp.int32),
            pltpu.VMEM((per_w,), jnp.int32),
            pltpu.SemaphoreType.DMA,
        ],
        compiler_params=pltpu.CompilerParams(use_tc_tiling_on_sc=False),
    )
    def k(tok_hbm, o2n_hbm, newtok_hbm, ci_hbm, tok_v, nt_v, ci_v, sem):
        wid = lax.axis_index("s") * _NC + lax.axis_index("c")
        base = wid * per_w
        pltpu.sync_copy(tok_hbm.at[pl.ds(base, per_w)], tok_v)
        pltpu.async_copy(o2n_hbm.at[tok_v], nt_v, sem).wait()

        def body(i, carry):
            nt = nt_v[pl.ds(i * _L, _L)]
            ci_v[pl.ds(i * _L, _L)] = jnp.maximum(nt - _KHOT, 0)
            return carry

        lax.fori_loop(0, per_w // _L, body, 0)
        pltpu.sync_copy(nt_v, newtok_hbm.at[pl.ds(base, per_w)])
        pltpu.sync_copy(ci_v, ci_hbm.at[pl.ds(base, per_w)])

    return k(tokens_flat, old_to_new)


def _pack_body(in_ref, *out_refs):
    for p in range(_NG):
        lo = jax.lax.bitcast_convert_type(in_ref[2 * p, :], jnp.int32)
        hi = jax.lax.bitcast_convert_type(in_ref[2 * p + 1, :], jnp.int32)
        lo16 = jax.lax.shift_right_logical(lo + 0x8000, 16)
        hi16 = jax.lax.shift_right_logical(hi + 0x8000, 16)
        out_refs[p][...] = jax.lax.shift_left(hi16, 16) | lo16


def _tc_pack(U_cold_T, v, rb):
    blk = _TBL // 4
    grid_k = (v + blk - 1) // blk
    return pl.pallas_call(
        _pack_body,
        grid=(grid_k,),
        in_specs=[pl.BlockSpec((2 * _NG, blk), lambda k, rb=rb: (rb, k))],
        out_specs=[pl.BlockSpec((blk,), lambda k: (k,))
                   for _ in range(_NG)],
        out_shape=[jax.ShapeDtypeStruct((_TBL,), jnp.int32)
                   for _ in range(_NG)],
    )(U_cold_T)


def _sc_gather(cold_idx, packed):
    n = cold_idx.shape[0]
    per_w = n // _NW
    mesh = plsc.VectorSubcoreMesh(core_axis_name="c", subcore_axis_name="s")

    @functools.partial(
        pl.kernel,
        out_type=tuple(jax.ShapeDtypeStruct((n,), jnp.int32)
                       for _ in range(_NG)),
        mesh=mesh,
        scratch_types=[
            pltpu.VMEM((per_w,), jnp.int32),
            pltpu.VMEM((_NG, per_w), jnp.int32),
            pltpu.SemaphoreType.DMA,
        ],
        compiler_params=pltpu.CompilerParams(use_tc_tiling_on_sc=False),
    )
    def k(ci_hbm, t0, t1, t2, t3, u0, u1, u2, u3, ci_v, ucP_v, sem):
        outs = (u0, u1, u2, u3)
        tables = (t0, t1, t2, t3)
        wid = lax.axis_index("s") * _NC + lax.axis_index("c")
        base = wid * per_w
        pltpu.sync_copy(ci_hbm.at[pl.ds(base, per_w)], ci_v)
        copies = [
            pltpu.async_copy(tables[p].at[ci_v], ucP_v.at[p], sem)
            for p in range(_NG)
        ]
        for c in copies:
            c.wait()
        for p in range(_NG):
            pltpu.sync_copy(ucP_v.at[p], outs[p].at[pl.ds(base, per_w)])

    return k(cold_idx, *packed)


def _tc_body(nt_ref, a0, a1, a2, a3, b0, b1, b2, b3,
             uhot_ref, bhot_ref, bcat_ref, out_ref):
    nt = nt_ref[0, 0, :]
    wa = jnp.stack([a0[...], a1[...], a2[...], a3[...]])
    wb = jnp.stack([b0[...], b1[...], b2[...], b3[...]])
    cat = jnp.concatenate([
        jax.lax.bitcast_convert_type(jax.lax.shift_left(wa, 16), jnp.float32),
        jax.lax.bitcast_convert_type(wa & jnp.int32(-65536), jnp.float32),
        jax.lax.bitcast_convert_type(jax.lax.shift_left(wb, 16), jnp.float32),
        jax.lax.bitcast_convert_type(wb & jnp.int32(-65536), jnp.float32),
    ], axis=0)
    coldT = lax.dot_general(bcat_ref[...], cat, (((0,), (0,)), ((), ())),
                            preferred_element_type=jnp.float32)
    out_ref[0] = coldT
    any_hot = jnp.min(nt) < _KHOT

    @pl.when(any_hot)
    def _():
        hot_tab = jnp.dot(uhot_ref[...], bhot_ref[...],
                          preferred_element_type=jnp.float32)
        ids = lax.broadcasted_iota(jnp.int32, (_KHOT, 1), 0)
        onehotT = (ids == nt[None, :]).astype(jnp.float32)
        hotT = lax.dot_general(hot_tab, onehotT,
                               (((0,), (0,)), ((), ())),
                               preferred_element_type=jnp.float32)
        is_hot = nt[None, :] < _KHOT
        out_ref[0] = jnp.where(is_hot, hotT, coldT)


def _tc_combine(new_tok_u, ucPa, ucPb, U_hot, B_hot, B_cat, n_rows, n_cols):
    blk = 4096
    kb = n_cols // blk
    nt3 = new_tok_u.reshape(n_rows, 1, n_cols)
    oned = pl.BlockSpec((blk,), lambda s, k: (s * kb + k,))
    return pl.pallas_call(
        _tc_body,
        grid=(n_rows, kb),
        in_specs=[
            pl.BlockSpec((1, 1, blk), lambda s, k: (s, 0, k)),
            oned, oned, oned, oned, oned, oned, oned, oned,
            pl.BlockSpec((_KHOT, _D), lambda s, k: (0, 0)),
            pl.BlockSpec((_D, _D), lambda s, k: (0, 0)),
            pl.BlockSpec((_RCOLD, _D), lambda s, k: (0, 0)),
        ],
        out_specs=pl.BlockSpec((1, _D, blk), lambda s, k: (s, 0, k)),
        out_shape=jax.ShapeDtypeStruct((n_rows, _D, n_cols), jnp.float32),
    )(nt3, *ucPa, *ucPb, U_hot, B_hot, B_cat)


def kernel(tokens, old_to_new, U_hot, U_cold, B_hot, B_cold):
    n_rows, n_cols = tokens.shape[1], tokens.shape[0]
    v = U_cold.shape[0]
    tok_u = jnp.transpose(tokens).reshape(-1)
    U_cold_T = jnp.transpose(U_cold)
    new_tok_u, cold_idx = _sc_map(tok_u, old_to_new)
    packed_a = _tc_pack(U_cold_T, v, 0)
    ucPa = _sc_gather(cold_idx, packed_a)
    packed_b = _tc_pack(U_cold_T, v, 1)
    ucPb = _sc_gather(cold_idx, packed_b)
    B_cat = B_cold[jnp.array([0, 2, 4, 6, 1, 3, 5, 7,
                              8, 10, 12, 14, 9, 11, 13, 15]), :]
    out_T = _tc_combine(new_tok_u, ucPa, ucPb, U_hot, B_hot, B_cat,
                        n_rows, n_cols)
    return jnp.transpose(out_T, (2, 0, 1))


# speedup vs baseline: 1.0193x; 1.0193x over previous
import functools

import jax
import jax.numpy as jnp
from jax import lax
from jax.experimental import pallas as pl
from jax.experimental.pallas import tpu as pltpu
from jax.experimental.pallas import tpu_sc as plsc

_KHOT = 128
_RCOLD = 16
_NPAIR = _RCOLD // 2
_NG = _NPAIR // 2
_D = 64
_NC, _NS, _L = 2, 16, 16
_NW = _NC * _NS
_TBL = 1 << 20


def _sc_map(tokens_flat, old_to_new):
    n = tokens_flat.shape[0]
    per_w = n // _NW
    mesh = plsc.VectorSubcoreMesh(core_axis_name="c", subcore_axis_name="s")

    @functools.partial(
        pl.kernel,
        out_type=(
            jax.ShapeDtypeStruct((n,), jnp.int32),
            jax.ShapeDtypeStruct((n,), jnp.int32),
        ),
        mesh=mesh,
        scratch_types=[
            pltpu.VMEM((per_w,), jnp.int32),
            pltpu.VMEM((per_w,), jnp.int32),
            pltpu.VMEM((per_w,), jnp.int32),
            pltpu.SemaphoreType.DMA,
        ],
        compiler_params=pltpu.CompilerParams(use_tc_tiling_on_sc=False),
    )
    def k(tok_hbm, o2n_hbm, newtok_hbm, ci_hbm, tok_v, nt_v, ci_v, sem):
        wid = lax.axis_index("s") * _NC + lax.axis_index("c")
        base = wid * per_w
        pltpu.sync_copy(tok_hbm.at[pl.ds(base, per_w)], tok_v)
        pltpu.async_copy(o2n_hbm.at[tok_v], nt_v, sem).wait()

        def body(i, carry):
            nt = nt_v[pl.ds(i * _L, _L)]
            ci_v[pl.ds(i * _L, _L)] = jnp.maximum(nt - _KHOT, 0)
            return carry

        lax.fori_loop(0, per_w // _L, body, 0)
        pltpu.sync_copy(nt_v, newtok_hbm.at[pl.ds(base, per_w)])
        pltpu.sync_copy(ci_v, ci_hbm.at[pl.ds(base, per_w)])

    return k(tokens_flat, old_to_new)


def _pack_body(in_ref, *out_refs):
    for p in range(_NG):
        lo = jax.lax.bitcast_convert_type(in_ref[2 * p, :], jnp.int32)
        hi = jax.lax.bitcast_convert_type(in_ref[2 * p + 1, :], jnp.int32)
        lo16 = jax.lax.shift_right_logical(lo + 0x8000, 16)
        hi16 = jax.lax.shift_right_logical(hi + 0x8000, 16)
        out_refs[p][...] = jax.lax.shift_left(hi16, 16) | lo16


def _tc_pack(U_cold_T, v, rb):
    blk = _TBL // 8
    grid_k = (v + blk - 1) // blk
    return pl.pallas_call(
        _pack_body,
        grid=(grid_k,),
        in_specs=[pl.BlockSpec((2 * _NG, blk), lambda k, rb=rb: (rb, k))],
        out_specs=[pl.BlockSpec((blk,), lambda k: (k,))
                   for _ in range(_NG)],
        out_shape=[jax.ShapeDtypeStruct((_TBL,), jnp.int32)
                   for _ in range(_NG)],
    )(U_cold_T)


def _sc_gather(cold_idx, packed):
    n = cold_idx.shape[0]
    per_w = n // _NW
    mesh = plsc.VectorSubcoreMesh(core_axis_name="c", subcore_axis_name="s")

    @functools.partial(
        pl.kernel,
        out_type=tuple(jax.ShapeDtypeStruct((n,), jnp.int32)
                       for _ in range(_NG)),
        mesh=mesh,
        scratch_types=[
            pltpu.VMEM((per_w,), jnp.int32),
            pltpu.VMEM((_NG, per_w), jnp.int32),
            pltpu.SemaphoreType.DMA,
        ],
        compiler_params=pltpu.CompilerParams(use_tc_tiling_on_sc=False),
    )
    def k(ci_hbm, t0, t1, t2, t3, u0, u1, u2, u3, ci_v, ucP_v, sem):
        outs = (u0, u1, u2, u3)
        tables = (t0, t1, t2, t3)
        wid = lax.axis_index("s") * _NC + lax.axis_index("c")
        base = wid * per_w
        pltpu.sync_copy(ci_hbm.at[pl.ds(base, per_w)], ci_v)
        copies = [
            pltpu.async_copy(tables[p].at[ci_v], ucP_v.at[p], sem)
            for p in range(_NG)
        ]
        for c in copies:
            c.wait()
        for p in range(_NG):
            pltpu.sync_copy(ucP_v.at[p], outs[p].at[pl.ds(base, per_w)])

    return k(cold_idx, *packed)


def _tc_body(nt_ref, a0, a1, a2, a3, b0, b1, b2, b3,
             uhot_ref, bhot_ref, bcat_ref, out_ref):
    nt = nt_ref[0, 0, :]
    wa = jnp.stack([a0[...], a1[...], a2[...], a3[...]])
    wb = jnp.stack([b0[...], b1[...], b2[...], b3[...]])
    cat = jnp.concatenate([
        jax.lax.bitcast_convert_type(jax.lax.shift_left(wa, 16), jnp.float32),
        jax.lax.bitcast_convert_type(wa & jnp.int32(-65536), jnp.float32),
        jax.lax.bitcast_convert_type(jax.lax.shift_left(wb, 16), jnp.float32),
        jax.lax.bitcast_convert_type(wb & jnp.int32(-65536), jnp.float32),
    ], axis=0)
    coldT = lax.dot_general(bcat_ref[...], cat, (((0,), (0,)), ((), ())),
                            preferred_element_type=jnp.float32)
    out_ref[0] = coldT
    any_hot = jnp.min(nt) < _KHOT

    @pl.when(any_hot)
    def _():
        hot_tab = jnp.dot(uhot_ref[...], bhot_ref[...],
                          preferred_element_type=jnp.float32)
        ids = lax.broadcasted_iota(jnp.int32, (_KHOT, 1), 0)
        onehotT = (ids == nt[None, :]).astype(jnp.float32)
        hotT = lax.dot_general(hot_tab, onehotT,
                               (((0,), (0,)), ((), ())),
                               preferred_element_type=jnp.float32)
        is_hot = nt[None, :] < _KHOT
        out_ref[0] = jnp.where(is_hot, hotT, coldT)


def _tc_combine(new_tok_u, ucPa, ucPb, U_hot, B_hot, B_cat, n_rows, n_cols):
    blk = 4096
    kb = n_cols // blk
    nt3 = new_tok_u.reshape(n_rows, 1, n_cols)
    oned = pl.BlockSpec((blk,), lambda s, k: (s * kb + k,))
    return pl.pallas_call(
        _tc_body,
        grid=(n_rows, kb),
        in_specs=[
            pl.BlockSpec((1, 1, blk), lambda s, k: (s, 0, k)),
            oned, oned, oned, oned, oned, oned, oned, oned,
            pl.BlockSpec((_KHOT, _D), lambda s, k: (0, 0)),
            pl.BlockSpec((_D, _D), lambda s, k: (0, 0)),
            pl.BlockSpec((_RCOLD, _D), lambda s, k: (0, 0)),
        ],
        out_specs=pl.BlockSpec((1, _D, blk), lambda s, k: (s, 0, k)),
        out_shape=jax.ShapeDtypeStruct((n_rows, _D, n_cols), jnp.float32),
    )(nt3, *ucPa, *ucPb, U_hot, B_hot, B_cat)


def kernel(tokens, old_to_new, U_hot, U_cold, B_hot, B_cold):
    n_rows, n_cols = tokens.shape[1], tokens.shape[0]
    v = U_cold.shape[0]
    tok_u = jnp.transpose(tokens).reshape(-1)
    U_cold_T = jnp.transpose(U_cold)
    new_tok_u, cold_idx = _sc_map(tok_u, old_to_new)
    packed_a = _tc_pack(U_cold_T, v, 0)
    ucPa = _sc_gather(cold_idx, packed_a)
    packed_b = _tc_pack(U_cold_T, v, 1)
    ucPb = _sc_gather(cold_idx, packed_b)
    B_cat = B_cold[jnp.array([0, 2, 4, 6, 1, 3, 5, 7,
                              8, 10, 12, 14, 9, 11, 13, 15]), :]
    out_T = _tc_combine(new_tok_u, ucPa, ucPb, U_hot, B_hot, B_cat,
                        n_rows, n_cols)
    return jnp.transpose(out_T, (2, 0, 1))
